# baseline (speedup 1.0000x reference)
import jax
import jax.numpy as jnp
from jax.experimental import pallas as pl
from jax.experimental.pallas import tpu as pltpu

_BM = 4


def _netvlad_body(r_ref, w_ref, b_ref, c_ref, o_ref):
    w = w_ref[...]
    bcol = b_ref[...]
    cent = c_ref[...]
    for i in range(_BM):
        rt = r_ref[0, i]
        rb = rt.astype(jnp.bfloat16)
        lt = jax.lax.dot_general(
            w, rb, (((1,), (0,)), ((), ())),
            preferred_element_type=jnp.float32,
        ) + bcol
        e = jnp.exp(lt)
        a = e / jnp.sum(e, axis=0, keepdims=True)
        v = jax.lax.dot_general(
            a.astype(jnp.bfloat16), rb, (((1,), (1,)), ((), ())),
            preferred_element_type=jnp.float32,
        )
        s = jnp.sum(a, axis=1, keepdims=True)
        o_ref[0, i] = v - s * cent


def kernel(R_seq, W, b, centroids, *, interpret=False):
    B, T, N, C = R_seq.shape
    K = centroids.shape[0]
    b2 = b.reshape(K, 1)
    rt = jnp.swapaxes(R_seq, 2, 3)
    tb = T // _BM
    out = pl.pallas_call(
        _netvlad_body,
        grid=(B * tb,),
        in_specs=[
            pl.BlockSpec((1, _BM, C, N), lambda i: (i // tb, i % tb, 0, 0)),
            pl.BlockSpec((K, C), lambda i: (0, 0)),
            pl.BlockSpec((K, 1), lambda i: (0, 0)),
            pl.BlockSpec((K, C), lambda i: (0, 0)),
        ],
        out_specs=pl.BlockSpec((1, _BM, K, C), lambda i: (i // tb, i % tb, 0, 0)),
        out_shape=jax.ShapeDtypeStruct((B, T, K, C), jnp.float32),
        compiler_params=pltpu.CompilerParams(
            dimension_semantics=("parallel",),
        ),
        name="netvlad_pool",
        interpret=interpret,
    )(rt, W, b2, centroids)
    return out


# speedup vs baseline: 1.5042x; 1.5042x over previous
import jax
import jax.numpy as jnp
from jax.experimental import pallas as pl
from jax.experimental.pallas import tpu as pltpu

_BM = 16


def _netvlad_body(r_ref, w_ref, b_ref, c_ref, o_ref):
    w = w_ref[...]
    bcol = b_ref[...]
    cent = c_ref[...]
    for i in range(_BM):
        rt = r_ref[0, i]
        rb = rt.astype(jnp.bfloat16)
        lt = jax.lax.dot_general(
            w, rb, (((1,), (0,)), ((), ())),
            preferred_element_type=jnp.float32,
        ) + bcol
        e = jnp.exp(lt)
        a = e / jnp.sum(e, axis=0, keepdims=True)
        v = jax.lax.dot_general(
            a.astype(jnp.bfloat16), rb, (((1,), (1,)), ((), ())),
            preferred_element_type=jnp.float32,
        )
        s = jnp.sum(a, axis=1, keepdims=True)
        o_ref[0, i] = v - s * cent


def kernel(R_seq, W, b, centroids, *, interpret=False):
    B, T, N, C = R_seq.shape
    K = centroids.shape[0]
    b2 = b.reshape(K, 1)
    rt = jnp.swapaxes(R_seq, 2, 3)
    tb = T // _BM
    out = pl.pallas_call(
        _netvlad_body,
        grid=(B * tb,),
        in_specs=[
            pl.BlockSpec((1, _BM, C, N), lambda i: (i // tb, i % tb, 0, 0)),
            pl.BlockSpec((K, C), lambda i: (0, 0)),
            pl.BlockSpec((K, 1), lambda i: (0, 0)),
            pl.BlockSpec((K, C), lambda i: (0, 0)),
        ],
        out_specs=pl.BlockSpec((1, _BM, K, C), lambda i: (i // tb, i % tb, 0, 0)),
        out_shape=jax.ShapeDtypeStruct((B, T, K, C), jnp.float32),
        compiler_params=pltpu.CompilerParams(
            dimension_semantics=("parallel",),
        ),
        name="netvlad_pool",
        interpret=interpret,
    )(rt, W, b2, centroids)
    return out


# speedup vs baseline: 1.5388x; 1.0230x over previous
import jax
import jax.numpy as jnp
from jax.experimental import pallas as pl
from jax.experimental.pallas import tpu as pltpu

_BM = 32


def _netvlad_body(r_ref, w_ref, b_ref, c_ref, o_ref):
    w = w_ref[...]
    bcol = b_ref[...]
    cent = c_ref[...]
    for i in range(_BM):
        rt = r_ref[0, i]
        rb = rt.astype(jnp.bfloat16)
        lt = jax.lax.dot_general(
            w, rb, (((1,), (0,)), ((), ())),
            preferred_element_type=jnp.float32,
        ) + bcol
        e = jnp.exp(lt)
        a = e / jnp.sum(e, axis=0, keepdims=True)
        v = jax.lax.dot_general(
            a.astype(jnp.bfloat16), rb, (((1,), (1,)), ((), ())),
            preferred_element_type=jnp.float32,
        )
        s = jnp.sum(a, axis=1, keepdims=True)
        o_ref[0, i] = v - s * cent


def kernel(R_seq, W, b, centroids, *, interpret=False):
    B, T, N, C = R_seq.shape
    K = centroids.shape[0]
    b2 = b.reshape(K, 1)
    rt = jnp.swapaxes(R_seq, 2, 3)
    tb = T // _BM
    out = pl.pallas_call(
        _netvlad_body,
        grid=(B * tb,),
        in_specs=[
            pl.BlockSpec((1, _BM, C, N), lambda i: (i // tb, i % tb, 0, 0)),
            pl.BlockSpec((K, C), lambda i: (0, 0)),
            pl.BlockSpec((K, 1), lambda i: (0, 0)),
            pl.BlockSpec((K, C), lambda i: (0, 0)),
        ],
        out_specs=pl.BlockSpec((1, _BM, K, C), lambda i: (i // tb, i % tb, 0, 0)),
        out_shape=jax.ShapeDtypeStruct((B, T, K, C), jnp.float32),
        compiler_params=pltpu.CompilerParams(
            dimension_semantics=("parallel",),
        ),
        name="netvlad_pool",
        interpret=interpret,
    )(rt, W, b2, centroids)
    return out


# speedup vs baseline: 1.5607x; 1.0143x over previous
import jax
import jax.numpy as jnp
from jax.experimental import pallas as pl
from jax.experimental.pallas import tpu as pltpu

_BM = 8
_NBUF = 4


def _slab_compute(rt, w, bcol, cent):
    rb = rt.astype(jnp.bfloat16)
    lt = jax.lax.dot_general(
        w, rb, (((1,), (0,)), ((), ())),
        preferred_element_type=jnp.float32,
    ) + bcol
    e = jnp.exp(lt)
    a = e / jnp.sum(e, axis=0, keepdims=True)
    v = jax.lax.dot_general(
        a.astype(jnp.bfloat16), rb, (((1,), (1,)), ((), ())),
        preferred_element_type=jnp.float32,
    )
    s = jnp.sum(a, axis=1, keepdims=True)
    return v - s * cent


def _netvlad_body(r_hbm, w_ref, b_ref, c_ref, o_ref, buf, sem):
    w = w_ref[...]
    bcol = b_ref[...]
    cent = c_ref[...]
    T = r_hbm.shape[1]
    tc = T // _BM
    n_chunks = r_hbm.shape[0] * tc

    def _start(j, slot):
        b_i = j // tc
        t0 = (j % tc) * _BM
        pltpu.make_async_copy(
            r_hbm.at[b_i, pl.ds(t0, _BM)], buf.at[slot], sem.at[slot]
        ).start()

    for k in range(_NBUF):
        _start(k, k)

    def _chunk(j, carry):
        slot = jax.lax.rem(j, _NBUF)
        b_i = j // tc
        t0 = (j % tc) * _BM
        pltpu.make_async_copy(
            r_hbm.at[b_i, pl.ds(t0, _BM)], buf.at[slot], sem.at[slot]
        ).wait()
        for i in range(_BM):
            o_ref[b_i, t0 + i] = _slab_compute(buf[slot, i], w, bcol, cent)

        @pl.when(j + _NBUF < n_chunks)
        def _():
            _start(j + _NBUF, slot)

        return carry

    jax.lax.fori_loop(0, n_chunks, _chunk, 0)


def kernel(R_seq, W, b, centroids, *, interpret=False):
    B, T, N, C = R_seq.shape
    K = centroids.shape[0]
    b2 = b.reshape(K, 1)
    rt = jnp.swapaxes(R_seq, 2, 3)
    out = pl.pallas_call(
        _netvlad_body,
        in_specs=[
            pl.BlockSpec(memory_space=pl.ANY),
            pl.BlockSpec((K, C), lambda: (0, 0)),
            pl.BlockSpec((K, 1), lambda: (0, 0)),
            pl.BlockSpec((K, C), lambda: (0, 0)),
        ],
        out_specs=pl.BlockSpec((B, T, K, C), lambda: (0, 0, 0, 0)),
        out_shape=jax.ShapeDtypeStruct((B, T, K, C), jnp.float32),
        scratch_shapes=[
            pltpu.VMEM((_NBUF, _BM, C, N), jnp.float32),
            pltpu.SemaphoreType.DMA((_NBUF,)),
        ],
        name="netvlad_pool",
        interpret=interpret,
    )(rt, W, b2, centroids)
    return out


# speedup vs baseline: 1.5765x; 1.0101x over previous
import jax
import jax.numpy as jnp
from jax.experimental import pallas as pl
from jax.experimental.pallas import tpu as pltpu

_BM = 16
_NBUF = 4


def _slab_compute(rt, w, bcol, cent):
    rb = rt.astype(jnp.bfloat16)
    lt = jax.lax.dot_general(
        w, rb, (((1,), (0,)), ((), ())),
        preferred_element_type=jnp.float32,
    ) + bcol
    e = jnp.exp(lt)
    a = e / jnp.sum(e, axis=0, keepdims=True)
    v = jax.lax.dot_general(
        a.astype(jnp.bfloat16), rb, (((1,), (1,)), ((), ())),
        preferred_element_type=jnp.float32,
    )
    s = jnp.sum(a, axis=1, keepdims=True)
    return v - s * cent


def _netvlad_body(r_hbm, w_ref, b_ref, c_ref, o_ref, buf, sem):
    w = w_ref[...]
    bcol = b_ref[...]
    cent = c_ref[...]
    T = r_hbm.shape[1]
    tc = T // _BM
    n_chunks = r_hbm.shape[0] * tc

    def _start(j, slot):
        b_i = j // tc
        t0 = (j % tc) * _BM
        pltpu.make_async_copy(
            r_hbm.at[b_i, pl.ds(t0, _BM)], buf.at[slot], sem.at[slot]
        ).start()

    for k in range(_NBUF):
        _start(k, k)

    def _chunk(j, carry):
        slot = jax.lax.rem(j, _NBUF)
        b_i = j // tc
        t0 = (j % tc) * _BM
        pltpu.make_async_copy(
            r_hbm.at[b_i, pl.ds(t0, _BM)], buf.at[slot], sem.at[slot]
        ).wait()
        for i in range(_BM):
            o_ref[b_i, t0 + i] = _slab_compute(buf[slot, i], w, bcol, cent)

        @pl.when(j + _NBUF < n_chunks)
        def _():
            _start(j + _NBUF, slot)

        return carry

    jax.lax.fori_loop(0, n_chunks, _chunk, 0)


def kernel(R_seq, W, b, centroids, *, interpret=False):
    B, T, N, C = R_seq.shape
    K = centroids.shape[0]
    b2 = b.reshape(K, 1)
    rt = jnp.swapaxes(R_seq, 2, 3)
    out = pl.pallas_call(
        _netvlad_body,
        in_specs=[
            pl.BlockSpec(memory_space=pl.ANY),
            pl.BlockSpec((K, C), lambda: (0, 0)),
            pl.BlockSpec((K, 1), lambda: (0, 0)),
            pl.BlockSpec((K, C), lambda: (0, 0)),
        ],
        out_specs=pl.BlockSpec((B, T, K, C), lambda: (0, 0, 0, 0)),
        out_shape=jax.ShapeDtypeStruct((B, T, K, C), jnp.float32),
        scratch_shapes=[
            pltpu.VMEM((_NBUF, _BM, C, N), jnp.float32),
            pltpu.SemaphoreType.DMA((_NBUF,)),
        ],
        name="netvlad_pool",
        interpret=interpret,
    )(rt, W, b2, centroids)
    return out
